# baseline (speedup 1.0000x reference)
import functools

import jax
import jax.numpy as jnp
from jax import lax
from jax.experimental import pallas as pl
from jax.experimental.pallas import tpu as pltpu
from jax.experimental.pallas import tpu_sc as plsc

N = 10000
D = 128
E = 320000
EPS = 1e-5

NC = 2
NS = 16
NW = NC * NS
EPT = E // NW
EB = 80
NEB = EPT // EB
CH = 80
NCHUNK = N // CH
CPT = -(-NCHUNK // NS)

RB = 1000
NRB = N // RB


def _sc_mesh():
  return plsc.VectorSubcoreMesh(
      core_axis_name="c", subcore_axis_name="s", num_cores=NC, num_subcores=NS
  )


def _deg_body(dst_hbm, out_hbm, idx_v, deg_v):
  c = lax.axis_index("c")
  s = lax.axis_index("s")
  wid = c * NS + s
  pltpu.sync_copy(dst_hbm.at[pl.ds(wid * EPT, EPT)], idx_v)

  def zero(i, carry):
    deg_v[pl.ds(i * 16, 16)] = jnp.zeros((16,), jnp.float32)
    return carry

  lax.fori_loop(0, N // 16, zero, 0)

  def body(i, carry):
    idx = idx_v[pl.ds(i * 16, 16)]
    cnt, last = plsc.scan_count(idx)
    plsc.addupdate_scatter(deg_v, [idx], cnt.astype(jnp.float32), mask=last)
    return carry

  lax.fori_loop(0, EPT // 16, body, 0)
  pltpu.sync_copy(deg_v, out_hbm.at[wid])


_deg_call = pl.kernel(
    _deg_body,
    out_type=jax.ShapeDtypeStruct((NW, N), jnp.float32),
    mesh=_sc_mesh(),
    scratch_types=[
        pltpu.VMEM((EPT,), jnp.int32),
        pltpu.VMEM((N,), jnp.float32),
    ],
    compiler_params=pltpu.CompilerParams(needs_layout_passes=False),
)


def _mmh_body(x_ref, w_ref, h_ref):
  h_ref[...] = jnp.dot(x_ref[...], w_ref[...],
                       preferred_element_type=jnp.float32)


_mmh_call = pl.pallas_call(
    _mmh_body,
    grid=(NRB,),
    in_specs=[
        pl.BlockSpec((RB, D), lambda i: (i, 0)),
        pl.BlockSpec((D, D), lambda i: (0, 0)),
    ],
    out_specs=pl.BlockSpec((RB, D), lambda i: (i, 0)),
    out_shape=jax.ShapeDtypeStruct((N, D), jnp.float32),
)


def _dis_body(degp_ref, dis_ref):
  deg = jnp.sum(degp_ref[...], axis=0) + 1.0
  dis_ref[...] = lax.rsqrt(deg)[:, None]


_dis_call = pl.pallas_call(
    _dis_body,
    grid=(1,),
    in_specs=[pl.BlockSpec((NW, N), lambda i: (0, 0))],
    out_specs=pl.BlockSpec((N, 1), lambda i: (0, 0)),
    out_shape=jax.ShapeDtypeStruct((N, 1), jnp.float32),
)


def _scale_body(dis_ref, h_ref, hs_ref):
  hs_ref[...] = h_ref[...] * dis_ref[...]


_scale_call = pl.pallas_call(
    _scale_body,
    grid=(NRB,),
    in_specs=[
        pl.BlockSpec((RB, 1), lambda i: (i, 0)),
        pl.BlockSpec((RB, D), lambda i: (i, 0)),
    ],
    out_specs=pl.BlockSpec((RB, D), lambda i: (i, 0)),
    out_shape=jax.ShapeDtypeStruct((N, D), jnp.float32),
)


def _agg_body(xs_hbm, src_hbm, dst_hbm, out_hbm, sidx1, didx2, buf_a, buf_b,
              shared, gs_a, gs_b, ss_a, ss_b):
  c = lax.axis_index("c")
  s = lax.axis_index("s")
  wid = c * NS + s

  stage_s = pltpu.async_copy(src_hbm.at[pl.ds(wid * EPT, EPT)], sidx1, gs_a)
  stage_d = pltpu.async_copy(dst_hbm.at[wid], didx2, gs_b)

  def zrow(i, carry):
    def zcol(k, carry2):
      buf_a[i, pl.ds(k * 16, 16)] = jnp.zeros((16,), jnp.float32)
      return carry2

    return lax.fori_loop(0, D // 16, zcol, carry)

  lax.fori_loop(0, CH, zrow, 0)
  stage_s.wait()
  stage_d.wait()

  def zcopy(j, carry):
    ch = s + j * NS

    @pl.when(ch < NCHUNK)
    def _():
      pltpu.sync_copy(buf_a, shared.at[pl.ds(ch * CH, CH)])

    return carry

  lax.fori_loop(0, CPT, zcopy, 0)
  plsc.subcore_barrier()

  def gather(i, buf, sem):
    return pltpu.async_copy(xs_hbm.at[sidx1.at[pl.ds(i * EB, EB)]], buf, sem)

  def gather_wait(i, buf, sem):
    pltpu.make_async_copy(
        xs_hbm.at[sidx1.at[pl.ds(i * EB, EB)]], buf, sem
    ).wait()

  def scatter(i, buf, sem):
    return pltpu.async_copy(buf, shared.at[didx2.at[i]], sem, add=True)

  def scatter_wait(i, buf, sem):
    pltpu.make_async_copy(buf, shared.at[didx2.at[i]], sem).wait()

  gather(0, buf_a, gs_a)

  def body(it, carry):
    i = 2 * it
    gather_wait(i, buf_a, gs_a)
    scatter(i, buf_a, ss_a)
    gather(i + 1, buf_b, gs_b)
    scatter_wait(i, buf_a, ss_a)
    gather(i + 2, buf_a, gs_a)
    gather_wait(i + 1, buf_b, gs_b)
    scatter(i + 1, buf_b, ss_b)
    scatter_wait(i + 1, buf_b, ss_b)
    return carry

  lax.fori_loop(0, (NEB - 1) // 2, body, 0)
  gather_wait(NEB - 1, buf_a, gs_a)
  pltpu.sync_copy(buf_a, shared.at[didx2.at[NEB - 1]], add=True)
  plsc.subcore_barrier()

  def orow(j, carry):
    ch = s + j * NS

    @pl.when(ch < NCHUNK)
    def _():
      r0 = ch * CH
      pltpu.sync_copy(shared.at[pl.ds(r0, CH)], out_hbm.at[c, pl.ds(r0, CH)])

    return carry

  lax.fori_loop(0, CPT, orow, 0)


_agg_call = pl.kernel(
    _agg_body,
    out_type=jax.ShapeDtypeStruct((NC, N, D), jnp.float32),
    mesh=_sc_mesh(),
    scratch_types=[
        pltpu.VMEM((EPT,), jnp.int32),
        pltpu.VMEM((NEB, EB), jnp.int32),
        pltpu.VMEM((EB, D), jnp.float32),
        pltpu.VMEM((EB, D), jnp.float32),
        pltpu.VMEM_SHARED((N, D), jnp.float32),
        pltpu.SemaphoreType.DMA,
        pltpu.SemaphoreType.DMA,
        pltpu.SemaphoreType.DMA,
        pltpu.SemaphoreType.DMA,
    ],
    compiler_params=pltpu.CompilerParams(needs_layout_passes=False),
)


def _mm_body(aggp_ref, hs_ref, dis_ref, b_ref, y_ref, s1_ref, s2_ref):
  i = pl.program_id(0)
  agg = aggp_ref[0] + aggp_ref[1]
  y = (agg + hs_ref[...]) * dis_ref[...] + b_ref[...]
  y_ref[...] = y
  ps = jnp.sum(y, axis=0, keepdims=True)
  pss = jnp.sum(y * y, axis=0, keepdims=True)

  @pl.when(i == 0)
  def _():
    s1_ref[...] = ps
    s2_ref[...] = pss

  @pl.when(i > 0)
  def _():
    s1_ref[...] += ps
    s2_ref[...] += pss


_mm_call = pl.pallas_call(
    _mm_body,
    grid=(NRB,),
    in_specs=[
        pl.BlockSpec((NC, RB, D), lambda i: (0, i, 0)),
        pl.BlockSpec((RB, D), lambda i: (i, 0)),
        pl.BlockSpec((RB, 1), lambda i: (i, 0)),
        pl.BlockSpec((1, D), lambda i: (0, 0)),
    ],
    out_specs=[
        pl.BlockSpec((RB, D), lambda i: (i, 0)),
        pl.BlockSpec((1, D), lambda i: (0, 0)),
        pl.BlockSpec((1, D), lambda i: (0, 0)),
    ],
    out_shape=[
        jax.ShapeDtypeStruct((N, D), jnp.float32),
        jax.ShapeDtypeStruct((1, D), jnp.float32),
        jax.ShapeDtypeStruct((1, D), jnp.float32),
    ],
)


def _bn_body(y_ref, s1_ref, s2_ref, gamma_ref, beta_ref, o_ref):
  mean = s1_ref[...] * (1.0 / N)
  var = s2_ref[...] * (1.0 / N) - mean * mean
  scale = gamma_ref[...] * lax.rsqrt(var + EPS)
  shift = beta_ref[...] - mean * scale
  o_ref[...] = jnp.maximum(y_ref[...] * scale + shift, 0.0)


_bn_call = pl.pallas_call(
    _bn_body,
    grid=(NRB,),
    in_specs=[
        pl.BlockSpec((RB, D), lambda i: (i, 0)),
        pl.BlockSpec((1, D), lambda i: (0, 0)),
        pl.BlockSpec((1, D), lambda i: (0, 0)),
        pl.BlockSpec((1, D), lambda i: (0, 0)),
        pl.BlockSpec((1, D), lambda i: (0, 0)),
    ],
    out_specs=pl.BlockSpec((RB, D), lambda i: (i, 0)),
    out_shape=jax.ShapeDtypeStruct((N, D), jnp.float32),
)


def kernel(x, edge_index, W, b, gamma, beta):
  src = edge_index[0].astype(jnp.int32)
  dst = edge_index[1].astype(jnp.int32)
  x = x.astype(jnp.float32)
  degp = _deg_call(dst)
  h = _mmh_call(x, W)
  dis = _dis_call(degp)
  hs = _scale_call(dis, h)
  aggp = _agg_call(hs, src, dst.reshape(NW, NEB, EB))
  y, s1, s2 = _mm_call(aggp, hs, dis, b.reshape(1, D))
  return _bn_call(y, s1, s2, gamma.reshape(1, D), beta.reshape(1, D))


# speedup vs baseline: 1.0708x; 1.0708x over previous
import functools

import jax
import jax.numpy as jnp
from jax import lax
from jax.experimental import pallas as pl
from jax.experimental.pallas import tpu as pltpu
from jax.experimental.pallas import tpu_sc as plsc

N = 10000
D = 128
E = 320000
EPS = 1e-5

NC = 2
NS = 16
NW = NC * NS
EPT = E // NW
EB = 80
NEB = EPT // EB
CH = 80
NCHUNK = N // CH
CPT = -(-NCHUNK // NS)

RB = 1000
NRB = N // RB


def _sc_mesh():
  return plsc.VectorSubcoreMesh(
      core_axis_name="c", subcore_axis_name="s", num_cores=NC, num_subcores=NS
  )


def _deg_body(dst_hbm, out_hbm, idx_v, deg_v):
  c = lax.axis_index("c")
  s = lax.axis_index("s")
  wid = c * NS + s
  pltpu.sync_copy(dst_hbm.at[pl.ds(wid * EPT, EPT)], idx_v)

  def zero(i, carry):
    deg_v[pl.ds(i * 16, 16)] = jnp.zeros((16,), jnp.float32)
    return carry

  lax.fori_loop(0, N // 16, zero, 0)

  def body(i, carry):
    idx = idx_v[pl.ds(i * 16, 16)]
    cnt, last = plsc.scan_count(idx)
    plsc.addupdate_scatter(deg_v, [idx], cnt.astype(jnp.float32), mask=last)
    return carry

  lax.fori_loop(0, EPT // 16, body, 0)
  pltpu.sync_copy(deg_v, out_hbm.at[wid])


_deg_call = pl.kernel(
    _deg_body,
    out_type=jax.ShapeDtypeStruct((NW, N), jnp.float32),
    mesh=_sc_mesh(),
    scratch_types=[
        pltpu.VMEM((EPT,), jnp.int32),
        pltpu.VMEM((N,), jnp.float32),
    ],
    compiler_params=pltpu.CompilerParams(needs_layout_passes=False),
)


def _mmh_body(x_ref, w_ref, h_ref):
  h_ref[...] = jnp.dot(x_ref[...], w_ref[...],
                       preferred_element_type=jnp.float32)


_mmh_call = pl.pallas_call(
    _mmh_body,
    grid=(NRB,),
    in_specs=[
        pl.BlockSpec((RB, D), lambda i: (i, 0)),
        pl.BlockSpec((D, D), lambda i: (0, 0)),
    ],
    out_specs=pl.BlockSpec((RB, D), lambda i: (i, 0)),
    out_shape=jax.ShapeDtypeStruct((N, D), jnp.float32),
)


def _pre_body(degp_ref, h_ref, dis_ref, hs_ref):
  deg = jnp.sum(degp_ref[...], axis=0) + 1.0
  dis = lax.rsqrt(deg)[:, None]
  dis_ref[...] = dis
  hs_ref[...] = h_ref[...] * dis


_pre_call = pl.pallas_call(
    _pre_body,
    grid=(1,),
    in_specs=[
        pl.BlockSpec((NW, N), lambda i: (0, 0)),
        pl.BlockSpec((N, D), lambda i: (0, 0)),
    ],
    out_specs=[
        pl.BlockSpec((N, 1), lambda i: (0, 0)),
        pl.BlockSpec((N, D), lambda i: (0, 0)),
    ],
    out_shape=[
        jax.ShapeDtypeStruct((N, 1), jnp.float32),
        jax.ShapeDtypeStruct((N, D), jnp.float32),
    ],
)


def _agg_body(xs_hbm, src_hbm, dst_hbm, out_hbm, sidx1, didx2, buf_a, buf_b,
              shared, gs_a, gs_b, ss_a, ss_b):
  c = lax.axis_index("c")
  s = lax.axis_index("s")
  wid = c * NS + s

  stage_s = pltpu.async_copy(src_hbm.at[pl.ds(wid * EPT, EPT)], sidx1, gs_a)
  stage_d = pltpu.async_copy(dst_hbm.at[wid], didx2, gs_b)

  def zrow(i, carry):
    def zcol(k, carry2):
      buf_a[i, pl.ds(k * 16, 16)] = jnp.zeros((16,), jnp.float32)
      return carry2

    return lax.fori_loop(0, D // 16, zcol, carry)

  lax.fori_loop(0, CH, zrow, 0)
  stage_s.wait()
  stage_d.wait()

  def zcopy(j, carry):
    ch = s + j * NS

    @pl.when(ch < NCHUNK)
    def _():
      pltpu.sync_copy(buf_a, shared.at[pl.ds(ch * CH, CH)])

    return carry

  lax.fori_loop(0, CPT, zcopy, 0)
  plsc.subcore_barrier()

  def gather(i, buf, sem):
    return pltpu.async_copy(xs_hbm.at[sidx1.at[pl.ds(i * EB, EB)]], buf, sem)

  def gather_wait(i, buf, sem):
    pltpu.make_async_copy(
        xs_hbm.at[sidx1.at[pl.ds(i * EB, EB)]], buf, sem
    ).wait()

  def scatter(i, buf, sem):
    return pltpu.async_copy(buf, shared.at[didx2.at[i]], sem, add=True)

  def scatter_wait(i, buf, sem):
    pltpu.make_async_copy(buf, shared.at[didx2.at[i]], sem).wait()

  gather(0, buf_a, gs_a)

  def body(it, carry):
    i = 2 * it
    gather_wait(i, buf_a, gs_a)
    scatter(i, buf_a, ss_a)
    gather(i + 1, buf_b, gs_b)
    scatter_wait(i, buf_a, ss_a)
    gather(i + 2, buf_a, gs_a)
    gather_wait(i + 1, buf_b, gs_b)
    scatter(i + 1, buf_b, ss_b)
    scatter_wait(i + 1, buf_b, ss_b)
    return carry

  lax.fori_loop(0, (NEB - 1) // 2, body, 0)
  gather_wait(NEB - 1, buf_a, gs_a)
  pltpu.sync_copy(buf_a, shared.at[didx2.at[NEB - 1]], add=True)
  plsc.subcore_barrier()

  def orow(j, carry):
    ch = s + j * NS

    @pl.when(ch < NCHUNK)
    def _():
      r0 = ch * CH
      pltpu.sync_copy(shared.at[pl.ds(r0, CH)], out_hbm.at[c, pl.ds(r0, CH)])

    return carry

  lax.fori_loop(0, CPT, orow, 0)


_agg_call = pl.kernel(
    _agg_body,
    out_type=jax.ShapeDtypeStruct((NC, N, D), jnp.float32),
    mesh=_sc_mesh(),
    scratch_types=[
        pltpu.VMEM((EPT,), jnp.int32),
        pltpu.VMEM((NEB, EB), jnp.int32),
        pltpu.VMEM((EB, D), jnp.float32),
        pltpu.VMEM((EB, D), jnp.float32),
        pltpu.VMEM_SHARED((N, D), jnp.float32),
        pltpu.SemaphoreType.DMA,
        pltpu.SemaphoreType.DMA,
        pltpu.SemaphoreType.DMA,
        pltpu.SemaphoreType.DMA,
    ],
    compiler_params=pltpu.CompilerParams(needs_layout_passes=False),
)


def _post_body(aggp_ref, hs_ref, dis_ref, b_ref, gamma_ref, beta_ref, o_ref,
               y_s, s1_s, s2_s):
  i = pl.program_id(0)

  @pl.when(i < NRB)
  def _():
    agg = aggp_ref[0] + aggp_ref[1]
    y = (agg + hs_ref[...]) * dis_ref[...] + b_ref[...]
    y_s[pl.ds(i * RB, RB), :] = y
    ps = jnp.sum(y, axis=0, keepdims=True)
    pss = jnp.sum(y * y, axis=0, keepdims=True)

    @pl.when(i == 0)
    def _():
      s1_s[...] = ps
      s2_s[...] = pss

    @pl.when(i > 0)
    def _():
      s1_s[...] += ps
      s2_s[...] += pss

  @pl.when(i >= NRB)
  def _():
    j = i - NRB
    mean = s1_s[...] * (1.0 / N)
    var = s2_s[...] * (1.0 / N) - mean * mean
    scale = gamma_ref[...] * lax.rsqrt(var + EPS)
    shift = beta_ref[...] - mean * scale
    y = y_s[pl.ds(j * RB, RB), :]
    o_ref[...] = jnp.maximum(y * scale + shift, 0.0)


_post_call = pl.pallas_call(
    _post_body,
    grid=(2 * NRB,),
    in_specs=[
        pl.BlockSpec((NC, RB, D), lambda i: (0, jnp.minimum(i, NRB - 1), 0)),
        pl.BlockSpec((RB, D), lambda i: (jnp.minimum(i, NRB - 1), 0)),
        pl.BlockSpec((RB, 1), lambda i: (jnp.minimum(i, NRB - 1), 0)),
        pl.BlockSpec((1, D), lambda i: (0, 0)),
        pl.BlockSpec((1, D), lambda i: (0, 0)),
        pl.BlockSpec((1, D), lambda i: (0, 0)),
    ],
    out_specs=pl.BlockSpec((RB, D), lambda i: (jnp.maximum(i - NRB, 0), 0)),
    out_shape=jax.ShapeDtypeStruct((N, D), jnp.float32),
    scratch_shapes=[
        pltpu.VMEM((N, D), jnp.float32),
        pltpu.VMEM((1, D), jnp.float32),
        pltpu.VMEM((1, D), jnp.float32),
    ],
)


def kernel(x, edge_index, W, b, gamma, beta):
  src = edge_index[0].astype(jnp.int32)
  dst = edge_index[1].astype(jnp.int32)
  x = x.astype(jnp.float32)
  degp = _deg_call(dst)
  h = _mmh_call(x, W)
  dis, hs = _pre_call(degp, h)
  aggp = _agg_call(hs, src, dst.reshape(NW, NEB, EB))
  return _post_call(aggp, hs, dis, b.reshape(1, D), gamma.reshape(1, D),
                    beta.reshape(1, D))
